# baseline (speedup 1.0000x reference)
import functools

import jax
import jax.numpy as jnp
from jax import lax
from jax.experimental import pallas as pl
from jax.experimental.pallas import tpu as pltpu
from jax.experimental.pallas import tpu_sc as plsc

_B = 4096
_L = 50
_K = 5
_D = 64
_V = 100000
_S = _B * _L * _K

_VODD = 3125
_VSH = 5

_TINY = 1.1754943508222875e-38


def _threefry2x32(x0, x1):
    k1 = jnp.uint32(0)
    k2 = jnp.uint32(42)
    ks2 = jnp.uint32(0 ^ 42 ^ 0x1BD11BDA)
    ks = (k1, k2, ks2)
    rots = ((13, 15, 26, 6), (17, 29, 16, 24))
    x0 = x0 + k1
    x1 = x1 + k2
    for i in range(5):
        for r in rots[i % 2]:
            x0 = x0 + x1
            x1 = (x1 << r) | (x1 >> (32 - r))
            x1 = x1 ^ x0
        x0 = x0 + ks[(i + 1) % 3]
        x1 = x1 + ks[(i + 2) % 3] + jnp.uint32(i + 1)
    return x0, x1


def _make_sample_kernel(n_samples, vocab, sb, vb, vodd, vsh):
    nv = -(-vocab // vb)
    nb = n_samples // sb

    def body(logits_ref, out_ref):
        step = pl.program_id(0)
        s0 = (step * sb).astype(jnp.uint32)
        srel = lax.broadcasted_iota(jnp.int32, (sb, vb), 0).astype(jnp.uint32)
        s_abs = srel + s0
        p = s_abs * jnp.uint32(vodd)
        hi0 = p >> (32 - vsh)
        lo0 = p << vsh
        vlane = lax.broadcasted_iota(jnp.int32, (sb, vb), 1)

        def vloop(t, carry):
            m, idx = carry
            v_i32 = vlane + t * vb
            v = v_i32.astype(jnp.uint32)
            lo = lo0 + v
            hi = hi0 + (lo < lo0).astype(jnp.uint32)
            b1, b2 = _threefry2x32(hi, lo)
            bits = b1 ^ b2
            fl = lax.bitcast_convert_type(
                (bits >> 9) | jnp.uint32(0x3F800000), jnp.float32) - 1.0
            u = jnp.maximum(fl, jnp.float32(_TINY))
            g = -jnp.log(-jnp.log(u))
            val = g + logits_ref[t, :][None, :]
            upd = val > m
            m = jnp.where(upd, val, m)
            idx = jnp.where(upd, v_i32, idx)
            return m, idx

        m0 = jnp.full((sb, vb), -jnp.inf, jnp.float32)
        i0 = jnp.zeros((sb, vb), jnp.int32)
        m, idx = lax.fori_loop(0, nv, vloop, (m0, i0))
        rowmax = jnp.max(m, axis=1, keepdims=True)
        masked = jnp.where(m == rowmax, idx, jnp.int32(0x7FFFFFFF))
        out_ref[0, 0, :] = jnp.min(masked, axis=1)

    return pl.pallas_call(
        body,
        grid=(nb,),
        in_specs=[pl.BlockSpec((nv, vb), lambda i: (0, 0))],
        out_specs=pl.BlockSpec((1, 1, sb), lambda i: (i, 0, 0)),
        out_shape=jax.ShapeDtypeStruct((nb, 1, sb), jnp.int32),
        compiler_params=pltpu.CompilerParams(
            dimension_semantics=("parallel",)),
    )


def _sample_negatives(logits_padded, n_samples=_S, vocab=_V, sb=32, vb=256,
                      vodd=_VODD, vsh=_VSH):
    call = _make_sample_kernel(n_samples, vocab, sb, vb, vodd, vsh)
    out = call(logits_padded)
    return out.reshape(n_samples)


def _pad_logits(logits, vocab, vb):
    nv = -(-vocab // vb)
    pad = nv * vb - vocab
    return jnp.pad(logits, (0, pad), constant_values=-jnp.inf).reshape(nv, vb)


def _make_gather_kernel(n_rows, n_ent, d):
    info = plsc.get_sparse_core_info()
    nw = info.num_cores * info.num_subcores
    ch = 128
    rw = n_rows // nw
    nch = rw // ch
    ew = n_ent // nw
    assert rw % ch == 0 and ew == ch
    mesh = plsc.VectorSubcoreMesh(core_axis_name="c", subcore_axis_name="s")

    @functools.partial(
        pl.kernel,
        mesh=mesh,
        out_type=(
            jax.ShapeDtypeStruct((n_rows, d), jnp.float32),
            jax.ShapeDtypeStruct((n_ent, d), jnp.float32),
        ),
        scratch_types=[
            pltpu.VMEM((ch,), jnp.int32),
            pltpu.VMEM((ch, d), jnp.float32),
            pltpu.SemaphoreType.DMA,
        ],
    )
    def gather_k(vtab_hbm, etab_hbm, idx_hbm, ids_hbm, outv_hbm, oute_hbm,
                 idx_v, rows_v, sem):
        wid = lax.axis_index("s") * info.num_cores + lax.axis_index("c")

        def chunk(t, carry):
            base = wid * rw + t * ch
            pltpu.sync_copy(idx_hbm.at[pl.ds(base, ch)], idx_v)
            pltpu.async_copy(vtab_hbm.at[idx_v], rows_v, sem).wait()
            pltpu.sync_copy(rows_v, outv_hbm.at[pl.ds(base, ch)])
            return carry

        lax.fori_loop(0, nch, chunk, 0)
        ebase = wid * ew
        pltpu.sync_copy(ids_hbm.at[pl.ds(ebase, ew)], idx_v)
        pltpu.async_copy(etab_hbm.at[idx_v], rows_v, sem).wait()
        pltpu.sync_copy(rows_v, oute_hbm.at[pl.ds(ebase, ew)])

    return gather_k


def _gather_rows(emb_v_table, emb_e_table, idx_all, ids_entity):
    k = _make_gather_kernel(idx_all.shape[0], ids_entity.shape[0],
                            emb_v_table.shape[1])
    return k(emb_v_table, emb_e_table, idx_all, ids_entity)


def _make_loss_kernel(b, npair, d, l, bb):
    nb = b // bb

    def body(rows_ref, e_ref, tp_ref, out_ref, acc_ref):
        step = pl.program_id(0)
        rows = rows_ref[...]
        e = e_ref[...]
        dot = jnp.sum(rows * e[:, None, :], axis=-1)
        terms = -jnp.log(jax.nn.sigmoid(dot) + 0.5)
        psum = jnp.sum(terms)
        plen = jnp.sum((tp_ref[...] != 0).astype(jnp.float32))

        @pl.when(step == 0)
        def _init():
            acc_ref[0] = psum
            acc_ref[1] = plen

        @pl.when(step != 0)
        def _acc():
            acc_ref[0] = acc_ref[0] + psum
            acc_ref[1] = acc_ref[1] + plen

        @pl.when(step == nb - 1)
        def _fin():
            out_ref[0] = acc_ref[0] / (jnp.float32(_K + 1) * acc_ref[1])

    return pl.pallas_call(
        body,
        grid=(nb,),
        in_specs=[
            pl.BlockSpec((bb, npair, d), lambda i: (i, 0, 0)),
            pl.BlockSpec((bb, d), lambda i: (i, 0)),
            pl.BlockSpec((bb, l), lambda i: (i, 0)),
        ],
        out_specs=pl.BlockSpec(memory_space=pltpu.MemorySpace.SMEM),
        out_shape=jax.ShapeDtypeStruct((1,), jnp.float32),
        scratch_shapes=[pltpu.SMEM((2,), jnp.float32)],
        compiler_params=pltpu.CompilerParams(
            dimension_semantics=("arbitrary",)),
    )


def _loss(rows, e_rows, token_pos, bb=32):
    b, npair, d = rows.shape
    call = _make_loss_kernel(b, npair, d, token_pos.shape[1], bb)
    return call(rows, e_rows, token_pos)[0]


def kernel(ids_entity, token_pos, emb_e_table, emb_v_table, distribution):
    logits = jnp.log(distribution)
    logits_padded = _pad_logits(logits, _V, 256)
    token_neg = _sample_negatives(logits_padded).reshape(_B, _L * _K)
    idx_all = jnp.concatenate(
        [token_pos.astype(jnp.int32), token_neg], axis=1).reshape(-1)
    vtab = jnp.pad(emb_v_table, ((0, 0), (0, 128 - _D)))
    etab = jnp.pad(emb_e_table, ((0, 0), (0, 128 - _D)))
    rows, e_rows = _gather_rows(
        vtab, etab, idx_all, ids_entity.astype(jnp.int32))
    rows = rows.reshape(_B, _L * (_K + 1), 128)
    return _loss(rows, e_rows, token_pos)


# speedup vs baseline: 1.1315x; 1.1315x over previous
import functools

import jax
import jax.numpy as jnp
from jax import lax
from jax.experimental import pallas as pl
from jax.experimental.pallas import tpu as pltpu
from jax.experimental.pallas import tpu_sc as plsc

_B = 4096
_L = 50
_K = 5
_D = 64
_V = 100000
_S = _B * _L * _K

_VODD = 3125
_VSH = 5

_TINY = 1.1754943508222875e-38


def _threefry2x32(x0, x1):
    _ks = (0, 42, 42 ^ 0x1BD11BDA)
    rots = ((13, 15, 26, 6), (17, 29, 16, 24))
    x1 = x1 + jnp.uint32(42)
    for i in range(5):
        for r in rots[i % 2]:
            x0 = x0 + x1
            x1 = (x1 << r) | (x1 >> (32 - r))
            x1 = x1 ^ x0
        c0 = _ks[(i + 1) % 3]
        c1 = (_ks[(i + 2) % 3] + i + 1) & 0xFFFFFFFF
        if c0:
            x0 = x0 + jnp.uint32(c0)
        x1 = x1 + jnp.uint32(c1)
    return x0, x1


_SB = 32
_VB = 256
_NU = 8


def _make_sample_kernel(n_samples, vocab, sb, vb, nu, vodd, vsh):
    nv = -(-(-(-vocab // vb)) // nu) * nu
    nb = n_samples // sb

    def body(logits_ref, out_ref):
        step = pl.program_id(0)
        s0 = (step * sb).astype(jnp.uint32)
        srel = lax.broadcasted_iota(jnp.int32, (sb, vb), 0).astype(jnp.uint32)
        s_abs = srel + s0
        p = s_abs * jnp.uint32(vodd)
        hi0 = p >> (32 - vsh)
        lo0 = p << vsh
        vlane = lax.broadcasted_iota(jnp.int32, (sb, vb), 1)

        def one_tile(t):
            v_i32 = vlane + t * vb
            v = v_i32.astype(jnp.uint32)
            lo = lo0 + v
            hi = hi0 + (lo < lo0).astype(jnp.uint32)
            b1, b2 = _threefry2x32(hi, lo)
            bits = b1 ^ b2
            fl = lax.bitcast_convert_type(
                (bits >> 9) | jnp.uint32(0x3F800000), jnp.float32) - 1.0
            u = jnp.maximum(fl, jnp.float32(_TINY))
            g = -jnp.log(-jnp.log(u))
            val = g + logits_ref[t, :][None, :]
            return val, v_i32

        def vloop(t, carry):
            m, idx = carry
            pairs = [one_tile(t * nu + j) for j in range(nu)]
            while len(pairs) > 1:
                merged = []
                for a in range(0, len(pairs) - 1, 2):
                    v0, i0_ = pairs[a]
                    v1, i1_ = pairs[a + 1]
                    take1 = v1 > v0
                    merged.append((jnp.where(take1, v1, v0),
                                   jnp.where(take1, i1_, i0_)))
                if len(pairs) % 2:
                    merged.append(pairs[-1])
                pairs = merged
            val, v_i32 = pairs[0]
            upd = val > m
            m = jnp.where(upd, val, m)
            idx = jnp.where(upd, v_i32, idx)
            return m, idx

        m0 = jnp.full((sb, vb), -jnp.inf, jnp.float32)
        i0 = jnp.zeros((sb, vb), jnp.int32)
        m, idx = lax.fori_loop(0, nv // nu, vloop, (m0, i0))
        rowmax = jnp.max(m, axis=1, keepdims=True)
        masked = jnp.where(m == rowmax, idx, jnp.int32(0x7FFFFFFF))
        out_ref[0, 0, :] = jnp.min(masked, axis=1)

    return pl.pallas_call(
        body,
        grid=(nb,),
        in_specs=[pl.BlockSpec((nv, vb), lambda i: (0, 0))],
        out_specs=pl.BlockSpec((1, 1, sb), lambda i: (i, 0, 0)),
        out_shape=jax.ShapeDtypeStruct((nb, 1, sb), jnp.int32),
        compiler_params=pltpu.CompilerParams(
            dimension_semantics=("parallel",)),
    )


def _sample_negatives(logits_padded, n_samples=_S, vocab=_V, sb=_SB, vb=_VB,
                      nu=_NU, vodd=_VODD, vsh=_VSH):
    call = _make_sample_kernel(n_samples, vocab, sb, vb, nu, vodd, vsh)
    out = call(logits_padded)
    return out.reshape(n_samples)


def _pad_logits(logits, vocab, vb=_VB, nu=_NU):
    nv = -(-(-(-vocab // vb)) // nu) * nu
    pad = nv * vb - vocab
    return jnp.pad(logits, (0, pad), constant_values=-jnp.inf).reshape(nv, vb)


def _make_gather_kernel(n_rows, n_ent, d):
    info = plsc.get_sparse_core_info()
    nw = info.num_cores * info.num_subcores
    ch = 128
    rw = n_rows // nw
    nch = rw // ch
    ew = n_ent // nw
    assert rw % ch == 0 and ew == ch
    mesh = plsc.VectorSubcoreMesh(core_axis_name="c", subcore_axis_name="s")

    @functools.partial(
        pl.kernel,
        mesh=mesh,
        out_type=(
            jax.ShapeDtypeStruct((n_rows, d), jnp.float32),
            jax.ShapeDtypeStruct((n_ent, d), jnp.float32),
        ),
        scratch_types=[
            pltpu.VMEM((ch,), jnp.int32),
            pltpu.VMEM((ch, d), jnp.float32),
            pltpu.SemaphoreType.DMA,
        ],
    )
    def gather_k(vtab_hbm, etab_hbm, idx_hbm, ids_hbm, outv_hbm, oute_hbm,
                 idx_v, rows_v, sem):
        wid = lax.axis_index("s") * info.num_cores + lax.axis_index("c")

        def chunk(t, carry):
            base = wid * rw + t * ch
            pltpu.sync_copy(idx_hbm.at[pl.ds(base, ch)], idx_v)
            pltpu.async_copy(vtab_hbm.at[idx_v], rows_v, sem).wait()
            pltpu.sync_copy(rows_v, outv_hbm.at[pl.ds(base, ch)])
            return carry

        lax.fori_loop(0, nch, chunk, 0)
        ebase = wid * ew
        pltpu.sync_copy(ids_hbm.at[pl.ds(ebase, ew)], idx_v)
        pltpu.async_copy(etab_hbm.at[idx_v], rows_v, sem).wait()
        pltpu.sync_copy(rows_v, oute_hbm.at[pl.ds(ebase, ew)])

    return gather_k


def _gather_rows(emb_v_table, emb_e_table, idx_all, ids_entity):
    k = _make_gather_kernel(idx_all.shape[0], ids_entity.shape[0],
                            emb_v_table.shape[1])
    return k(emb_v_table, emb_e_table, idx_all, ids_entity)


def _make_loss_kernel(b, npair, d, l, bb):
    nb = b // bb

    def body(rows_ref, e_ref, tp_ref, out_ref, acc_ref):
        step = pl.program_id(0)
        rows = rows_ref[...]
        e = e_ref[...]
        dot = jnp.sum(rows * e[:, None, :], axis=-1)
        terms = -jnp.log(jax.nn.sigmoid(dot) + 0.5)
        psum = jnp.sum(terms)
        plen = jnp.sum((tp_ref[...] != 0).astype(jnp.float32))

        @pl.when(step == 0)
        def _init():
            acc_ref[0] = psum
            acc_ref[1] = plen

        @pl.when(step != 0)
        def _acc():
            acc_ref[0] = acc_ref[0] + psum
            acc_ref[1] = acc_ref[1] + plen

        @pl.when(step == nb - 1)
        def _fin():
            out_ref[0] = acc_ref[0] / (jnp.float32(_K + 1) * acc_ref[1])

    return pl.pallas_call(
        body,
        grid=(nb,),
        in_specs=[
            pl.BlockSpec((bb, npair, d), lambda i: (i, 0, 0)),
            pl.BlockSpec((bb, d), lambda i: (i, 0)),
            pl.BlockSpec((bb, l), lambda i: (i, 0)),
        ],
        out_specs=pl.BlockSpec(memory_space=pltpu.MemorySpace.SMEM),
        out_shape=jax.ShapeDtypeStruct((1,), jnp.float32),
        scratch_shapes=[pltpu.SMEM((2,), jnp.float32)],
        compiler_params=pltpu.CompilerParams(
            dimension_semantics=("arbitrary",)),
    )


def _loss(rows, e_rows, token_pos, bb=32):
    b, npair, d = rows.shape
    call = _make_loss_kernel(b, npair, d, token_pos.shape[1], bb)
    return call(rows, e_rows, token_pos)[0]


def kernel(ids_entity, token_pos, emb_e_table, emb_v_table, distribution):
    logits = jnp.log(distribution)
    logits_padded = _pad_logits(logits, _V)
    token_neg = _sample_negatives(logits_padded).reshape(_B, _L * _K)
    idx_all = jnp.concatenate(
        [token_pos.astype(jnp.int32), token_neg], axis=1).reshape(-1)
    vtab = jnp.pad(emb_v_table, ((0, 0), (0, 128 - _D)))
    etab = jnp.pad(emb_e_table, ((0, 0), (0, 128 - _D)))
    rows, e_rows = _gather_rows(
        vtab, etab, idx_all, ids_entity.astype(jnp.int32))
    rows = rows.reshape(_B, _L * (_K + 1), 128)
    return _loss(rows, e_rows, token_pos)


# speedup vs baseline: 1.1701x; 1.0341x over previous
import functools

import jax
import jax.numpy as jnp
from jax import lax
from jax.experimental import pallas as pl
from jax.experimental.pallas import tpu as pltpu
from jax.experimental.pallas import tpu_sc as plsc

_B = 4096
_L = 50
_K = 5
_D = 64
_V = 100000
_S = _B * _L * _K

_VODD = 3125
_VSH = 5

_TINY = 1.1754943508222875e-38


def _threefry2x32(x0, x1):
    _ks = (0, 42, 42 ^ 0x1BD11BDA)
    rots = ((13, 15, 26, 6), (17, 29, 16, 24))
    x1 = x1 + jnp.uint32(42)
    for i in range(5):
        for r in rots[i % 2]:
            x0 = x0 + x1
            x1 = (x1 << r) | (x1 >> (32 - r))
            x1 = x1 ^ x0
        c0 = _ks[(i + 1) % 3]
        c1 = (_ks[(i + 2) % 3] + i + 1) & 0xFFFFFFFF
        if c0:
            x0 = x0 + jnp.uint32(c0)
        x1 = x1 + jnp.uint32(c1)
    return x0, x1


_SB = 32
_VB = 256
_NU = 8


def _make_sample_kernel(n_samples, vocab, sb, vb, nu, vodd, vsh):
    nv = -(-(-(-vocab // vb)) // nu) * nu
    nb = n_samples // sb

    def body(logits_ref, out_ref):
        step = pl.program_id(0)
        s0 = (step * sb).astype(jnp.uint32)
        srel = lax.broadcasted_iota(jnp.int32, (sb, vb), 0).astype(jnp.uint32)
        s_abs = srel + s0
        p = s_abs * jnp.uint32(vodd)
        hi0 = p >> (32 - vsh)
        lo0 = p << vsh
        vlane = lax.broadcasted_iota(jnp.int32, (sb, vb), 1)

        def one_tile(t):
            v_i32 = vlane + t * vb
            v = v_i32.astype(jnp.uint32)
            lo = lo0 + v
            hi = hi0 + (lo < lo0).astype(jnp.uint32)
            b1, b2 = _threefry2x32(hi, lo)
            bits = b1 ^ b2
            fl = lax.bitcast_convert_type(
                (bits >> 9) | jnp.uint32(0x3F800000), jnp.float32) - 1.0
            val = jnp.log2(fl) * logits_ref[t, :][None, :]
            return val, v_i32

        def vloop(t, carry):
            m, idx = carry
            pairs = [one_tile(t * nu + j) for j in range(nu)]
            while len(pairs) > 1:
                merged = []
                for a in range(0, len(pairs) - 1, 2):
                    v0, i0_ = pairs[a]
                    v1, i1_ = pairs[a + 1]
                    take1 = v1 < v0
                    merged.append((jnp.where(take1, v1, v0),
                                   jnp.where(take1, i1_, i0_)))
                if len(pairs) % 2:
                    merged.append(pairs[-1])
                pairs = merged
            val, v_i32 = pairs[0]
            upd = val < m
            m = jnp.where(upd, val, m)
            idx = jnp.where(upd, v_i32, idx)
            return m, idx

        m0 = jnp.full((sb, vb), jnp.inf, jnp.float32)
        i0 = jnp.zeros((sb, vb), jnp.int32)
        m, idx = lax.fori_loop(0, nv // nu, vloop, (m0, i0))
        rowmin = jnp.min(m, axis=1, keepdims=True)
        masked = jnp.where(m == rowmin, idx, jnp.int32(0x7FFFFFFF))
        out_ref[0, 0, :] = jnp.min(masked, axis=1)

    return pl.pallas_call(
        body,
        grid=(nb,),
        in_specs=[pl.BlockSpec((nv, vb), lambda i: (0, 0))],
        out_specs=pl.BlockSpec((1, 1, sb), lambda i: (i, 0, 0)),
        out_shape=jax.ShapeDtypeStruct((nb, 1, sb), jnp.int32),
        compiler_params=pltpu.CompilerParams(
            dimension_semantics=("parallel",)),
    )


def _sample_negatives(logits_padded, n_samples=_S, vocab=_V, sb=_SB, vb=_VB,
                      nu=_NU, vodd=_VODD, vsh=_VSH):
    call = _make_sample_kernel(n_samples, vocab, sb, vb, nu, vodd, vsh)
    out = call(logits_padded)
    return out.reshape(n_samples)


def _pad_weights(distribution, vocab, vb=_VB, nu=_NU):
    nv = -(-(-(-vocab // vb)) // nu) * nu
    pad = nv * vb - vocab
    c2 = -1.0 / distribution
    return jnp.pad(c2, (0, pad), constant_values=-jnp.inf).reshape(nv, vb)


def _make_gather_kernel(n_rows, n_ent, d):
    info = plsc.get_sparse_core_info()
    nw = info.num_cores * info.num_subcores
    ch = 128
    rw = n_rows // nw
    nch = rw // ch
    ew = n_ent // nw
    assert rw % ch == 0 and ew == ch
    mesh = plsc.VectorSubcoreMesh(core_axis_name="c", subcore_axis_name="s")

    @functools.partial(
        pl.kernel,
        mesh=mesh,
        out_type=(
            jax.ShapeDtypeStruct((n_rows, d), jnp.float32),
            jax.ShapeDtypeStruct((n_ent, d), jnp.float32),
        ),
        scratch_types=[
            pltpu.VMEM((ch,), jnp.int32),
            pltpu.VMEM((ch, d), jnp.float32),
            pltpu.SemaphoreType.DMA,
        ],
    )
    def gather_k(vtab_hbm, etab_hbm, idx_hbm, ids_hbm, outv_hbm, oute_hbm,
                 idx_v, rows_v, sem):
        wid = lax.axis_index("s") * info.num_cores + lax.axis_index("c")

        def chunk(t, carry):
            base = wid * rw + t * ch
            pltpu.sync_copy(idx_hbm.at[pl.ds(base, ch)], idx_v)
            pltpu.async_copy(vtab_hbm.at[idx_v], rows_v, sem).wait()
            pltpu.sync_copy(rows_v, outv_hbm.at[pl.ds(base, ch)])
            return carry

        lax.fori_loop(0, nch, chunk, 0)
        ebase = wid * ew
        pltpu.sync_copy(ids_hbm.at[pl.ds(ebase, ew)], idx_v)
        pltpu.async_copy(etab_hbm.at[idx_v], rows_v, sem).wait()
        pltpu.sync_copy(rows_v, oute_hbm.at[pl.ds(ebase, ew)])

    return gather_k


def _gather_rows(emb_v_table, emb_e_table, idx_all, ids_entity):
    k = _make_gather_kernel(idx_all.shape[0], ids_entity.shape[0],
                            emb_v_table.shape[1])
    return k(emb_v_table, emb_e_table, idx_all, ids_entity)


def _make_loss_kernel(b, npair, d, l, bb):
    nb = b // bb

    def body(rows_ref, e_ref, tp_ref, out_ref, acc_ref):
        step = pl.program_id(0)
        rows = rows_ref[...]
        e = e_ref[...]
        dot = jnp.sum(rows * e[:, None, :], axis=-1)
        terms = -jnp.log(jax.nn.sigmoid(dot) + 0.5)
        psum = jnp.sum(terms)
        plen = jnp.sum((tp_ref[...] != 0).astype(jnp.float32))

        @pl.when(step == 0)
        def _init():
            acc_ref[0] = psum
            acc_ref[1] = plen

        @pl.when(step != 0)
        def _acc():
            acc_ref[0] = acc_ref[0] + psum
            acc_ref[1] = acc_ref[1] + plen

        @pl.when(step == nb - 1)
        def _fin():
            out_ref[0] = acc_ref[0] / (jnp.float32(_K + 1) * acc_ref[1])

    return pl.pallas_call(
        body,
        grid=(nb,),
        in_specs=[
            pl.BlockSpec((bb, npair, d), lambda i: (i, 0, 0)),
            pl.BlockSpec((bb, d), lambda i: (i, 0)),
            pl.BlockSpec((bb, l), lambda i: (i, 0)),
        ],
        out_specs=pl.BlockSpec(memory_space=pltpu.MemorySpace.SMEM),
        out_shape=jax.ShapeDtypeStruct((1,), jnp.float32),
        scratch_shapes=[pltpu.SMEM((2,), jnp.float32)],
        compiler_params=pltpu.CompilerParams(
            dimension_semantics=("arbitrary",)),
    )


def _loss(rows, e_rows, token_pos, bb=32):
    b, npair, d = rows.shape
    call = _make_loss_kernel(b, npair, d, token_pos.shape[1], bb)
    return call(rows, e_rows, token_pos)[0]


def kernel(ids_entity, token_pos, emb_e_table, emb_v_table, distribution):
    weights_padded = _pad_weights(distribution, _V)
    token_neg = _sample_negatives(weights_padded).reshape(_B, _L * _K)
    idx_all = jnp.concatenate(
        [token_pos.astype(jnp.int32), token_neg], axis=1).reshape(-1)
    vtab = jnp.pad(emb_v_table, ((0, 0), (0, 128 - _D)))
    etab = jnp.pad(emb_e_table, ((0, 0), (0, 128 - _D)))
    rows, e_rows = _gather_rows(
        vtab, etab, idx_all, ids_entity.astype(jnp.int32))
    rows = rows.reshape(_B, _L * (_K + 1), 128)
    return _loss(rows, e_rows, token_pos)
